# baseline (device time: 116815 ns/iter reference)
import functools

import jax
import jax.numpy as jnp
from jax import lax
from jax.experimental import pallas as pl
from jax.experimental.pallas import tpu as pltpu

N_DEV = 8
M_PER = 512
K = 4096
N_TOT = 8192
N_PER = 1024
KC = 512
N_CHUNK = N_TOT // KC

E4M3_MAX = 448.0


def _body(x_ref, w_hbm, out_hbm,
          w_buf, partial, stage, q_send, q_recv, amax_buf,
          w_sems, out_sems, send_sems, recv_sems, ax_send_sems,
          ax_recv_sems):
    my = lax.axis_index("i")

    barrier_sem = pltpu.get_barrier_semaphore()
    for k in range(1, N_DEV):
        dst = lax.rem(my + k, N_DEV)
        pl.semaphore_signal(barrier_sem, inc=1, device_id=(dst,),
                            device_id_type=pl.DeviceIdType.MESH)
    pl.semaphore_wait(barrier_sem, N_DEV - 1)

    def _fetch(c, slot):
        cp = pltpu.make_async_copy(
            src_ref=w_hbm.at[:, pl.ds(c * KC, KC)],
            dst_ref=w_buf.at[slot],
            sem=w_sems.at[slot],
        )
        cp.start()
        return cp

    cps = [None, None]
    cps[0] = _fetch(0, 0)
    am = jnp.float32(0.0)
    for c in range(N_CHUNK):
        slot = c % 2
        if c + 1 < N_CHUNK:
            cps[(c + 1) % 2] = _fetch(c + 1, (c + 1) % 2)
        cps[slot].wait()
        blk = jnp.dot(x_ref[...], w_buf[slot],
                      preferred_element_type=jnp.float32)
        partial[:, pl.ds(c * KC, KC)] = blk
        am = jnp.maximum(am, jnp.max(jnp.abs(blk)))

    amax_buf[pl.ds(my, 1), :] = jnp.full((1, 128), am, jnp.float32)
    ax_sends = []
    for k in range(1, N_DEV):
        dst = lax.rem(my + k, N_DEV)
        r = pltpu.make_async_remote_copy(
            src_ref=amax_buf.at[pl.ds(my, 1)],
            dst_ref=amax_buf.at[pl.ds(my, 1)],
            send_sem=ax_send_sems.at[k],
            recv_sem=ax_recv_sems.at[my],
            device_id=(dst,),
            device_id_type=pl.DeviceIdType.MESH,
        )
        r.start()
        ax_sends.append(r)
    for k in range(1, N_DEV):
        src = lax.rem(my + N_DEV - k, N_DEV)
        ax_recv = pltpu.make_async_remote_copy(
            src_ref=amax_buf.at[pl.ds(src, 1)],
            dst_ref=amax_buf.at[pl.ds(src, 1)],
            send_sem=ax_send_sems.at[0],
            recv_sem=ax_recv_sems.at[src],
            device_id=(src,),
            device_id_type=pl.DeviceIdType.MESH,
        )
        ax_recv.wait_recv()
    g_amax = jnp.max(amax_buf[:, 0])
    scale = g_amax / E4M3_MAX
    inv_scale = E4M3_MAX / g_amax

    blk_sends = []
    for k in range(1, N_DEV):
        dst = lax.rem(my + k, N_DEV)
        q_send[:, pl.ds(dst * N_PER, N_PER)] = (
            partial[:, pl.ds(dst * N_PER, N_PER)] * inv_scale
        ).astype(jnp.float8_e4m3fn)
        r = pltpu.make_async_remote_copy(
            src_ref=q_send.at[:, pl.ds(dst * N_PER, N_PER)],
            dst_ref=q_recv.at[pl.ds(my * M_PER, M_PER), :],
            send_sem=send_sems.at[k],
            recv_sem=recv_sems.at[my],
            device_id=(dst,),
            device_id_type=pl.DeviceIdType.MESH,
        )
        r.start()
        blk_sends.append(r)

    out_cps = [None, None]

    def _store(rows, vals, slot):
        if out_cps[slot] is not None:
            out_cps[slot].wait()
        stage[slot] = vals
        cp = pltpu.make_async_copy(
            src_ref=stage.at[slot],
            dst_ref=out_hbm.at[pl.ds(rows, M_PER), :],
            sem=out_sems.at[slot],
        )
        cp.start()
        out_cps[slot] = cp

    own_q = (partial[:, pl.ds(my * N_PER, N_PER)] * inv_scale
             ).astype(jnp.float8_e4m3fn)
    _store(my * M_PER, own_q.astype(jnp.float32) * scale, 0)

    for k in range(1, N_DEV):
        src = lax.rem(my + N_DEV - k, N_DEV)
        recv = pltpu.make_async_remote_copy(
            src_ref=q_send.at[:, pl.ds(0, N_PER)],
            dst_ref=q_recv.at[pl.ds(src * M_PER, M_PER), :],
            send_sem=send_sems.at[0],
            recv_sem=recv_sems.at[src],
            device_id=(src,),
            device_id_type=pl.DeviceIdType.MESH,
        )
        recv.wait_recv()
        vals = (q_recv[pl.ds(src * M_PER, M_PER), :].astype(jnp.float32)
                * scale)
        _store(src * M_PER, vals, k % 2)

    out_cps[0].wait()
    out_cps[1].wait()
    for r in ax_sends:
        r.wait_send()
    for r in blk_sends:
        r.wait_send()

    @functools.partial(pl.run_scoped, exit_sem=pltpu.SemaphoreType.REGULAR)
    def _(exit_sem):
        for k in range(1, N_DEV):
            dst = lax.rem(my + k, N_DEV)
            pl.semaphore_signal(exit_sem, inc=1, device_id=(dst,),
                                device_id_type=pl.DeviceIdType.MESH)
        pl.semaphore_wait(exit_sem, N_DEV - 1)


def kernel(x, w_mat):
    return pl.pallas_call(
        _body,
        in_specs=[
            pl.BlockSpec(memory_space=pltpu.VMEM),
            pl.BlockSpec(memory_space=pl.ANY),
        ],
        out_specs=pl.BlockSpec(memory_space=pl.ANY),
        out_shape=jax.ShapeDtypeStruct((N_DEV * M_PER, N_PER), jnp.float32),
        scratch_shapes=[
            pltpu.VMEM((2, K, KC), jnp.float32),
            pltpu.VMEM((M_PER, N_TOT), jnp.float32),
            pltpu.VMEM((2, M_PER, N_PER), jnp.float32),
            pltpu.VMEM((M_PER, N_TOT), jnp.float8_e4m3fn),
            pltpu.VMEM((N_DEV * M_PER, N_PER), jnp.float8_e4m3fn),
            pltpu.VMEM((N_DEV, 128), jnp.float32),
            pltpu.SemaphoreType.DMA((2,)),
            pltpu.SemaphoreType.DMA((2,)),
            pltpu.SemaphoreType.DMA((N_DEV,)),
            pltpu.SemaphoreType.DMA((N_DEV,)),
            pltpu.SemaphoreType.DMA((N_DEV,)),
            pltpu.SemaphoreType.DMA((N_DEV,)),
        ],
        compiler_params=pltpu.CompilerParams(
            collective_id=0, vmem_limit_bytes=100 * 1024 * 1024),
    )(x, w_mat)


# device time: 110350 ns/iter; 1.0586x vs baseline; 1.0586x over previous
import functools

import jax
import jax.numpy as jnp
from jax import lax
from jax.experimental import pallas as pl
from jax.experimental.pallas import tpu as pltpu

N_DEV = 8
M_PER = 512
K = 4096
N_TOT = 8192
N_PER = 1024
KC = 512
N_CHUNK = N_TOT // KC
N_BF = 2

E4M3_MAX = 448.0


def _body(x_ref, w_hbm, bf_out, scale_ref,
          w_buf, partial, bf_send, q_send, q_recv, amax_buf,
          w_sems, send_sems, recv_sems, bf_send_sems, bf_recv_sems,
          ax_send_sems, ax_recv_sems):
    my = lax.axis_index("i")

    barrier_sem = pltpu.get_barrier_semaphore()
    for k in range(1, N_DEV):
        dst = lax.rem(my + k, N_DEV)
        pl.semaphore_signal(barrier_sem, inc=1, device_id=(dst,),
                            device_id_type=pl.DeviceIdType.MESH)
    pl.semaphore_wait(barrier_sem, N_DEV - 1)

    def _chunk_off(c):
        b = c // 2 + 1
        dst = lax.rem(my + b, N_DEV)
        return dst * N_PER + (c % 2) * KC

    def _fetch(c, slot):
        cp = pltpu.make_async_copy(
            src_ref=w_hbm.at[:, pl.ds(_chunk_off(c), KC)],
            dst_ref=w_buf.at[slot],
            sem=w_sems.at[slot],
        )
        cp.start()
        return cp

    cps = [None, None]
    cps[0] = _fetch(0, 0)
    am = jnp.float32(0.0)
    bf_sends = []
    for b in range(1, N_DEV + 1):
        dst = lax.rem(my + b, N_DEV)
        for h in range(2):
            c = 2 * (b - 1) + h
            slot = c % 2
            if c + 1 < N_CHUNK:
                cps[(c + 1) % 2] = _fetch(c + 1, (c + 1) % 2)
            cps[slot].wait()
            blk = jnp.dot(x_ref[...], w_buf[slot],
                          preferred_element_type=jnp.float32)
            am = jnp.maximum(am, jnp.max(jnp.abs(blk)))
            if b <= N_BF:
                bf_send[:, pl.ds((b - 1) * N_PER + h * KC, KC)] = (
                    blk.astype(jnp.bfloat16))
            else:
                bf_col = (b - 1 - N_BF) * N_PER + h * KC
                partial[:, pl.ds(bf_col, KC)] = blk
        if b <= N_BF:
            r = pltpu.make_async_remote_copy(
                src_ref=bf_send.at[:, pl.ds((b - 1) * N_PER, N_PER)],
                dst_ref=bf_out.at[pl.ds(my * M_PER, M_PER), :],
                send_sem=bf_send_sems.at[b],
                recv_sem=bf_recv_sems.at[my],
                device_id=(dst,),
                device_id_type=pl.DeviceIdType.MESH,
            )
            r.start()
            bf_sends.append(r)

    amax_buf[pl.ds(my, 1), :] = jnp.full((1, 128), am, jnp.float32)
    ax_sends = []
    for k in range(1, N_DEV):
        dst = lax.rem(my + k, N_DEV)
        r = pltpu.make_async_remote_copy(
            src_ref=amax_buf.at[pl.ds(my, 1)],
            dst_ref=amax_buf.at[pl.ds(my, 1)],
            send_sem=ax_send_sems.at[k],
            recv_sem=ax_recv_sems.at[my],
            device_id=(dst,),
            device_id_type=pl.DeviceIdType.MESH,
        )
        r.start()
        ax_sends.append(r)
    for k in range(1, N_DEV):
        src = lax.rem(my + N_DEV - k, N_DEV)
        ax_recv = pltpu.make_async_remote_copy(
            src_ref=amax_buf.at[pl.ds(src, 1)],
            dst_ref=amax_buf.at[pl.ds(src, 1)],
            send_sem=ax_send_sems.at[0],
            recv_sem=ax_recv_sems.at[src],
            device_id=(src,),
            device_id_type=pl.DeviceIdType.MESH,
        )
        ax_recv.wait_recv()
    g_amax = jnp.max(amax_buf[:, 0])
    scale = g_amax / E4M3_MAX
    inv_scale = E4M3_MAX / g_amax
    scale_ref[0, 0] = scale
    scale_ref[0, 1] = inv_scale

    blk_sends = []
    for b in range(N_BF + 1, N_DEV):
        dst = lax.rem(my + b, N_DEV)
        pcol = (b - 1 - N_BF) * N_PER
        q_send[:, pl.ds(pcol, N_PER)] = (
            partial[:, pl.ds(pcol, N_PER)] * inv_scale
        ).astype(jnp.float8_e4m3fn)
        r = pltpu.make_async_remote_copy(
            src_ref=q_send.at[:, pl.ds(pcol, N_PER)],
            dst_ref=q_recv.at[pl.ds(my * M_PER, M_PER), :],
            send_sem=send_sems.at[b],
            recv_sem=recv_sems.at[my],
            device_id=(dst,),
            device_id_type=pl.DeviceIdType.MESH,
        )
        r.start()
        blk_sends.append(r)

    own = (partial[:, pl.ds((N_DEV - 1 - N_BF) * N_PER, N_PER)]
           * inv_scale).astype(jnp.float8_e4m3fn)
    bf_out[pl.ds(my * M_PER, M_PER), :] = (
        own.astype(jnp.float32) * scale).astype(jnp.bfloat16)

    for b in range(N_BF + 1, N_DEV):
        src = lax.rem(my + N_DEV - b, N_DEV)
        recv = pltpu.make_async_remote_copy(
            src_ref=q_send.at[:, pl.ds(0, N_PER)],
            dst_ref=q_recv.at[pl.ds(src * M_PER, M_PER), :],
            send_sem=send_sems.at[0],
            recv_sem=recv_sems.at[src],
            device_id=(src,),
            device_id_type=pl.DeviceIdType.MESH,
        )
        recv.wait_recv()
        bf_out[pl.ds(src * M_PER, M_PER), :] = (
            q_recv[pl.ds(src * M_PER, M_PER), :].astype(jnp.float32)
            * scale).astype(jnp.bfloat16)

    for b in range(1, N_BF + 1):
        src = lax.rem(my + N_DEV - b, N_DEV)
        recv = pltpu.make_async_remote_copy(
            src_ref=bf_send.at[:, pl.ds(0, N_PER)],
            dst_ref=bf_out.at[pl.ds(src * M_PER, M_PER), :],
            send_sem=bf_send_sems.at[0],
            recv_sem=bf_recv_sems.at[src],
            device_id=(src,),
            device_id_type=pl.DeviceIdType.MESH,
        )
        recv.wait_recv()

    for r in ax_sends:
        r.wait_send()
    for r in bf_sends:
        r.wait_send()
    for r in blk_sends:
        r.wait_send()

    @functools.partial(pl.run_scoped, exit_sem=pltpu.SemaphoreType.REGULAR)
    def _(exit_sem):
        for k in range(1, N_DEV):
            dst = lax.rem(my + k, N_DEV)
            pl.semaphore_signal(exit_sem, inc=1, device_id=(dst,),
                                device_id_type=pl.DeviceIdType.MESH)
        pl.semaphore_wait(exit_sem, N_DEV - 1)


def kernel(x, w_mat):
    bf, sc = pl.pallas_call(
        _body,
        in_specs=[
            pl.BlockSpec(memory_space=pltpu.VMEM),
            pl.BlockSpec(memory_space=pl.ANY),
        ],
        out_specs=[
            pl.BlockSpec(memory_space=pltpu.VMEM),
            pl.BlockSpec(memory_space=pltpu.SMEM),
        ],
        out_shape=[
            jax.ShapeDtypeStruct((N_DEV * M_PER, N_PER), jnp.bfloat16),
            jax.ShapeDtypeStruct((1, 2), jnp.float32),
        ],
        scratch_shapes=[
            pltpu.VMEM((2, K, KC), jnp.float32),
            pltpu.VMEM((M_PER, (N_DEV - N_BF) * N_PER),
                       jnp.float32),
            pltpu.VMEM((M_PER, N_BF * N_PER), jnp.bfloat16),
            pltpu.VMEM((M_PER, (N_DEV - 1 - N_BF) * N_PER),
                       jnp.float8_e4m3fn),
            pltpu.VMEM((N_DEV * M_PER, N_PER), jnp.float8_e4m3fn),
            pltpu.VMEM((N_DEV, 128), jnp.float32),
            pltpu.SemaphoreType.DMA((2,)),
            pltpu.SemaphoreType.DMA((N_DEV,)),
            pltpu.SemaphoreType.DMA((N_DEV,)),
            pltpu.SemaphoreType.DMA((N_DEV,)),
            pltpu.SemaphoreType.DMA((N_DEV,)),
            pltpu.SemaphoreType.DMA((N_DEV,)),
            pltpu.SemaphoreType.DMA((N_DEV,)),
        ],
        compiler_params=pltpu.CompilerParams(
            collective_id=0, vmem_limit_bytes=100 * 1024 * 1024),
    )(x, w_mat)
    scale, inv_scale = sc[0, 0], sc[0, 1]
    s = jnp.clip(bf.astype(jnp.float32) * inv_scale, -E4M3_MAX, E4M3_MAX)
    q = s.astype(jnp.float8_e4m3fn)
    return q.astype(jnp.float32) * scale


# device time: 108399 ns/iter; 1.0776x vs baseline; 1.0180x over previous
import functools

import jax
import jax.numpy as jnp
from jax import lax
from jax.experimental import pallas as pl
from jax.experimental.pallas import tpu as pltpu

N_DEV = 8
M_PER = 512
K = 4096
N_TOT = 8192
N_PER = 1024
KC = 512
N_CHUNK = N_TOT // KC

E4M3_MAX = 448.0


def _body(x_ref, w_hbm, q_out, scale_ref,
          w_buf, partial, stage_unused, q_send, amax_buf,
          w_sems, copy_sems, send_sems, recv_sems, ax_send_sems,
          ax_recv_sems):
    my = lax.axis_index("i")

    barrier_sem = pltpu.get_barrier_semaphore()
    for k in range(1, N_DEV):
        dst = lax.rem(my + k, N_DEV)
        pl.semaphore_signal(barrier_sem, inc=1, device_id=(dst,),
                            device_id_type=pl.DeviceIdType.MESH)
    pl.semaphore_wait(barrier_sem, N_DEV - 1)

    def _fetch(c, slot):
        cp = pltpu.make_async_copy(
            src_ref=w_hbm.at[:, pl.ds(c * KC, KC)],
            dst_ref=w_buf.at[slot],
            sem=w_sems.at[slot],
        )
        cp.start()
        return cp

    cps = [None, None]
    cps[0] = _fetch(0, 0)
    am = jnp.float32(0.0)
    for c in range(N_CHUNK):
        slot = c % 2
        if c + 1 < N_CHUNK:
            cps[(c + 1) % 2] = _fetch(c + 1, (c + 1) % 2)
        cps[slot].wait()
        blk = jnp.dot(x_ref[...], w_buf[slot],
                      preferred_element_type=jnp.float32)
        partial[:, pl.ds(c * KC, KC)] = blk
        am = jnp.maximum(am, jnp.max(jnp.abs(blk)))

    amax_buf[pl.ds(my, 1), :] = jnp.full((1, 128), am, jnp.float32)
    ax_sends = []
    for k in range(1, N_DEV):
        dst = lax.rem(my + k, N_DEV)
        r = pltpu.make_async_remote_copy(
            src_ref=amax_buf.at[pl.ds(my, 1)],
            dst_ref=amax_buf.at[pl.ds(my, 1)],
            send_sem=ax_send_sems.at[k],
            recv_sem=ax_recv_sems.at[my],
            device_id=(dst,),
            device_id_type=pl.DeviceIdType.MESH,
        )
        r.start()
        ax_sends.append(r)
    for k in range(1, N_DEV):
        src = lax.rem(my + N_DEV - k, N_DEV)
        ax_recv = pltpu.make_async_remote_copy(
            src_ref=amax_buf.at[pl.ds(src, 1)],
            dst_ref=amax_buf.at[pl.ds(src, 1)],
            send_sem=ax_send_sems.at[0],
            recv_sem=ax_recv_sems.at[src],
            device_id=(src,),
            device_id_type=pl.DeviceIdType.MESH,
        )
        ax_recv.wait_recv()
    g_amax = jnp.max(amax_buf[:, 0])
    scale_ref[0, 0] = g_amax / E4M3_MAX
    inv_scale = E4M3_MAX / g_amax

    blk_sends = []
    for k in range(1, N_DEV):
        dst = lax.rem(my + k, N_DEV)
        q_send[:, pl.ds(dst * N_PER, N_PER)] = (
            partial[:, pl.ds(dst * N_PER, N_PER)] * inv_scale
        ).astype(jnp.float8_e4m3fn)
        r = pltpu.make_async_remote_copy(
            src_ref=q_send.at[:, pl.ds(dst * N_PER, N_PER)],
            dst_ref=q_out.at[pl.ds(my * M_PER, M_PER), :],
            send_sem=send_sems.at[k],
            recv_sem=recv_sems.at[my],
            device_id=(dst,),
            device_id_type=pl.DeviceIdType.MESH,
        )
        r.start()
        blk_sends.append(r)

    q_out[pl.ds(my * M_PER, M_PER), :] = (
        partial[:, pl.ds(my * N_PER, N_PER)] * inv_scale
    ).astype(jnp.float8_e4m3fn)

    for k in range(1, N_DEV):
        src = lax.rem(my + N_DEV - k, N_DEV)
        recv = pltpu.make_async_remote_copy(
            src_ref=q_send.at[:, pl.ds(0, N_PER)],
            dst_ref=q_out.at[pl.ds(src * M_PER, M_PER), :],
            send_sem=send_sems.at[0],
            recv_sem=recv_sems.at[src],
            device_id=(src,),
            device_id_type=pl.DeviceIdType.MESH,
        )
        recv.wait_recv()

    for r in ax_sends:
        r.wait_send()
    for r in blk_sends:
        r.wait_send()

    @functools.partial(pl.run_scoped, exit_sem=pltpu.SemaphoreType.REGULAR)
    def _(exit_sem):
        for k in range(1, N_DEV):
            dst = lax.rem(my + k, N_DEV)
            pl.semaphore_signal(exit_sem, inc=1, device_id=(dst,),
                                device_id_type=pl.DeviceIdType.MESH)
        pl.semaphore_wait(exit_sem, N_DEV - 1)


def kernel(x, w_mat):
    q, scale = pl.pallas_call(
        _body,
        in_specs=[
            pl.BlockSpec(memory_space=pltpu.VMEM),
            pl.BlockSpec(memory_space=pl.ANY),
        ],
        out_specs=[
            pl.BlockSpec(memory_space=pltpu.VMEM),
            pl.BlockSpec(memory_space=pltpu.SMEM),
        ],
        out_shape=[
            jax.ShapeDtypeStruct((N_DEV * M_PER, N_PER), jnp.float8_e4m3fn),
            jax.ShapeDtypeStruct((1, 1), jnp.float32),
        ],
        scratch_shapes=[
            pltpu.VMEM((2, K, KC), jnp.float32),
            pltpu.VMEM((M_PER, N_TOT), jnp.float32),
            pltpu.VMEM((8, 128), jnp.float32),
            pltpu.VMEM((M_PER, N_TOT), jnp.float8_e4m3fn),
            pltpu.VMEM((N_DEV, 128), jnp.float32),
            pltpu.SemaphoreType.DMA((2,)),
            pltpu.SemaphoreType.DMA((2,)),
            pltpu.SemaphoreType.DMA((N_DEV,)),
            pltpu.SemaphoreType.DMA((N_DEV,)),
            pltpu.SemaphoreType.DMA((N_DEV,)),
            pltpu.SemaphoreType.DMA((N_DEV,)),
        ],
        compiler_params=pltpu.CompilerParams(
            collective_id=0, vmem_limit_bytes=100 * 1024 * 1024),
    )(x, w_mat)
    return q.astype(jnp.float32) * scale[0, 0]
